# baseline (speedup 1.0000x reference)
import functools

import jax
import jax.numpy as jnp
import numpy as np
from jax import lax
from jax.experimental import pallas as pl
from jax.experimental.pallas import tpu as pltpu
from jax.experimental.pallas import tpu_sc as plsc

_D = 64
_SEQ = 200
_BATCH = 4096
_N = _BATCH * _SEQ
_NW = 32
_PER_W = _N // _NW
_CHUNK = 40
_CHUNKS_PER_W = _PER_W // _CHUNK
_POS_ROWS_PER_CHUNK = _CHUNK


def _pos_encoding(max_len, d_model):
    angle = jnp.arange(d_model, dtype=jnp.float32)
    angle = 10000.0 ** (2.0 * (angle / d_model))
    angle = jnp.arange(max_len, dtype=jnp.float32)[:, None] / angle
    values = jnp.stack([jnp.sin(angle[:, 0::2]), jnp.cos(angle[:, 1::2])], axis=2)
    return jnp.reshape(values, (values.shape[0], -1)).astype(jnp.float32)


def _sc_embed(seq_hbm, pos_hbm, table_hbm, out_hbm, pos_v, idx_v, rows_v, sem):
    wid = lax.axis_index("s") * 2 + lax.axis_index("c")
    base = wid * _PER_W
    pltpu.sync_copy(pos_hbm, pos_v)

    def chunk_body(c, _):
        off = base + c * _CHUNK
        pltpu.sync_copy(seq_hbm.at[pl.ds(off, _CHUNK)], idx_v)
        pltpu.async_copy(table_hbm.at[idx_v], rows_v, sem).wait()
        prow = lax.rem(c, 5) * _CHUNK

        def row_body(j, _):
            for i in range(_D // 16):
                sl = pl.ds(i * 16, 16)
                rows_v[j, sl] = rows_v[j, sl] * 8.0 + pos_v[prow + j, sl]
            return 0

        lax.fori_loop(0, _CHUNK, row_body, 0)
        pltpu.sync_copy(rows_v, out_hbm.at[pl.ds(off, _CHUNK), :])
        return 0

    lax.fori_loop(0, _CHUNKS_PER_W, chunk_body, 0)


@jax.jit
def _embed(sequences, table):
    pos = _pos_encoding(_SEQ, _D)
    seq_flat = sequences.reshape(_N).astype(jnp.int32)
    mesh = plsc.VectorSubcoreMesh(core_axis_name="c", subcore_axis_name="s")
    out = pl.kernel(
        _sc_embed,
        out_type=jax.ShapeDtypeStruct((_N, _D), jnp.float32),
        mesh=mesh,
        scratch_types=[
            pltpu.VMEM((_SEQ, _D), jnp.float32),
            pltpu.VMEM((_CHUNK,), jnp.int32),
            pltpu.VMEM((_CHUNK, _D), jnp.float32),
            pltpu.SemaphoreType.DMA,
        ],
        compiler_params=pltpu.CompilerParams(use_tc_tiling_on_sc=False),
    )(seq_flat, pos, table)
    return out.reshape(_BATCH, _SEQ, _D)


def kernel(sequences, table):
    return _embed(sequences, table)


# speedup vs baseline: 1.4429x; 1.4429x over previous
import functools

import jax
import jax.numpy as jnp
import numpy as np
from jax import lax
from jax.experimental import pallas as pl
from jax.experimental.pallas import tpu as pltpu
from jax.experimental.pallas import tpu_sc as plsc

_D = 64
_SEQ = 200
_BATCH = 4096
_N = _BATCH * _SEQ
_NW = 32
_PER_W = _N // _NW
_CHUNK = 40
_NCHUNK = _PER_W // _CHUNK
_NBUF = 4
_NITER = _NCHUNK // _NBUF


def _pos_encoding(max_len, d_model):
    angle = jnp.arange(d_model, dtype=jnp.float32)
    angle = 10000.0 ** (2.0 * (angle / d_model))
    angle = jnp.arange(max_len, dtype=jnp.float32)[:, None] / angle
    values = jnp.stack([jnp.sin(angle[:, 0::2]), jnp.cos(angle[:, 1::2])], axis=2)
    return jnp.reshape(values, (values.shape[0], -1)).astype(jnp.float32)


def _sc_embed(seq_hbm, pos_hbm, table_hbm, out_hbm, pos_v, idx_v, rows_g,
              rows_w, gsems, wsems):
    wid = lax.axis_index("s") * 2 + lax.axis_index("c")
    base = wid * _NCHUNK
    pltpu.sync_copy(pos_hbm, pos_v)
    pltpu.sync_copy(seq_hbm.at[pl.ds(base, _NCHUNK), :], idx_v)

    def gather(c, b):
        pltpu.async_copy(table_hbm.at[idx_v.at[c]], rows_g[b], gsems[b])

    def write(c, b):
        pltpu.async_copy(rows_w[b],
                         out_hbm.at[pl.ds((base + c) * _CHUNK, _CHUNK), :],
                         wsems[b])

    for b in range(_NBUF):
        gather(b, b)

    def ring_iter(g, _):
        for b in range(_NBUF):
            c = g * _NBUF + b
            pltpu.make_async_copy(table_hbm.at[idx_v.at[c]], rows_g[b],
                                  gsems[b]).wait()
            @pl.when(g > 0)
            def _drain():
                pltpu.make_async_copy(
                    rows_w[b],
                    out_hbm.at[pl.ds((base + c - _NBUF) * _CHUNK, _CHUNK), :],
                    wsems[b]).wait()

            prow = lax.rem(c, 5) * _CHUNK

            def row_body(j, _):
                for i in range(_D // 16):
                    sl = pl.ds(i * 16, 16)
                    rows_w[b][j, sl] = rows_g[b][j, sl] * 8.0 + pos_v[prow + j, sl]
                return 0

            lax.fori_loop(0, _CHUNK, row_body, 0, unroll=4)

            @pl.when(g + 1 < _NITER)
            def _next():
                gather(c + _NBUF, b)

            write(c, b)
        return 0

    lax.fori_loop(0, _NITER, ring_iter, 0)
    for b in range(_NBUF):
        c = _NCHUNK - _NBUF + b
        pltpu.make_async_copy(
            rows_w[b], out_hbm.at[pl.ds((base + c) * _CHUNK, _CHUNK), :],
            wsems[b]).wait()


@jax.jit
def _embed(sequences, table):
    pos = _pos_encoding(_SEQ, _D)
    seq2 = sequences.reshape(_N // _CHUNK, _CHUNK).astype(jnp.int32)
    mesh = plsc.VectorSubcoreMesh(core_axis_name="c", subcore_axis_name="s")
    out = pl.kernel(
        _sc_embed,
        out_type=jax.ShapeDtypeStruct((_N, _D), jnp.float32),
        mesh=mesh,
        scratch_types=[
            pltpu.VMEM((_SEQ, _D), jnp.float32),
            pltpu.VMEM((_NCHUNK, _CHUNK), jnp.int32),
            [pltpu.VMEM((_CHUNK, _D), jnp.float32) for _ in range(_NBUF)],
            [pltpu.VMEM((_CHUNK, _D), jnp.float32) for _ in range(_NBUF)],
            [pltpu.SemaphoreType.DMA for _ in range(_NBUF)],
            [pltpu.SemaphoreType.DMA for _ in range(_NBUF)],
        ],
        compiler_params=pltpu.CompilerParams(use_tc_tiling_on_sc=False),
    )(seq2, pos, table)
    return out.reshape(_BATCH, _SEQ, _D)


def kernel(sequences, table):
    return _embed(sequences, table)


# speedup vs baseline: 1.8964x; 1.3142x over previous
import functools

import jax
import jax.numpy as jnp
import numpy as np
from jax import lax
from jax.experimental import pallas as pl
from jax.experimental.pallas import tpu as pltpu
from jax.experimental.pallas import tpu_sc as plsc

_D = 64
_SEQ = 200
_BATCH = 4096
_N = _BATCH * _SEQ
_NW = 32
_PER_W = _N // _NW
_CHUNK = 40
_NCHUNK = _PER_W // _CHUNK
_NBUF = 4
_NITER = _NCHUNK // _NBUF


def _pos_encoding(max_len, d_model):
    angle = jnp.arange(d_model, dtype=jnp.float32)
    angle = 10000.0 ** (2.0 * (angle / d_model))
    angle = jnp.arange(max_len, dtype=jnp.float32)[:, None] / angle
    values = jnp.stack([jnp.sin(angle[:, 0::2]), jnp.cos(angle[:, 1::2])], axis=2)
    return jnp.reshape(values, (values.shape[0], -1)).astype(jnp.float32)


def _sc_embed(seq_hbm, pos_hbm, table_hbm, out_hbm, pos_v, idx_v, rows_g,
              rows_w, gsems, wsems):
    wid = lax.axis_index("s") * 2 + lax.axis_index("c")
    base = wid * _NCHUNK
    pltpu.sync_copy(pos_hbm, pos_v)
    pltpu.sync_copy(seq_hbm.at[pl.ds(base, _NCHUNK), :], idx_v)

    def gather(c, b):
        pltpu.async_copy(table_hbm.at[idx_v.at[c]], rows_g[b], gsems[b])

    def write(c, b):
        pltpu.async_copy(rows_w[b],
                         out_hbm.at[pl.ds((base + c) * _CHUNK, _CHUNK), :],
                         wsems[b])

    for b in range(_NBUF):
        gather(b, b)

    def ring_iter(g, _):
        for b in range(_NBUF):
            c = g * _NBUF + b
            pltpu.make_async_copy(table_hbm.at[idx_v.at[c]], rows_g[b],
                                  gsems[b]).wait()
            @pl.when(g > 0)
            def _drain():
                pltpu.make_async_copy(
                    rows_w[b],
                    out_hbm.at[pl.ds((base + c - _NBUF) * _CHUNK, _CHUNK), :],
                    wsems[b]).wait()

            prow = lax.rem(c, 5) * _CHUNK

            @plsc.parallel_loop(0, _CHUNK, unroll=8)
            def _rows(j):
                for i in range(_D // 16):
                    sl = pl.ds(i * 16, 16)
                    rows_w[b][j, sl] = rows_g[b][j, sl] * 8.0 + pos_v[prow + j, sl]

            @pl.when(g + 1 < _NITER)
            def _next():
                gather(c + _NBUF, b)

            write(c, b)
        return 0

    lax.fori_loop(0, _NITER, ring_iter, 0)
    for b in range(_NBUF):
        c = _NCHUNK - _NBUF + b
        pltpu.make_async_copy(
            rows_w[b], out_hbm.at[pl.ds((base + c) * _CHUNK, _CHUNK), :],
            wsems[b]).wait()


@jax.jit
def _embed(sequences, table):
    pos = _pos_encoding(_SEQ, _D)
    seq2 = sequences.reshape(_N // _CHUNK, _CHUNK).astype(jnp.int32)
    mesh = plsc.VectorSubcoreMesh(core_axis_name="c", subcore_axis_name="s")
    out = pl.kernel(
        _sc_embed,
        out_type=jax.ShapeDtypeStruct((_N, _D), jnp.float32),
        mesh=mesh,
        scratch_types=[
            pltpu.VMEM((_SEQ, _D), jnp.float32),
            pltpu.VMEM((_NCHUNK, _CHUNK), jnp.int32),
            [pltpu.VMEM((_CHUNK, _D), jnp.float32) for _ in range(_NBUF)],
            [pltpu.VMEM((_CHUNK, _D), jnp.float32) for _ in range(_NBUF)],
            [pltpu.SemaphoreType.DMA for _ in range(_NBUF)],
            [pltpu.SemaphoreType.DMA for _ in range(_NBUF)],
        ],
        compiler_params=pltpu.CompilerParams(use_tc_tiling_on_sc=False),
    )(seq2, pos, table)
    return out.reshape(_BATCH, _SEQ, _D)


def kernel(sequences, table):
    return _embed(sequences, table)
